# baseline (speedup 1.0000x reference)
import functools

import jax
import jax.numpy as jnp
from jax import lax
from jax.experimental import pallas as pl
from jax.experimental.pallas import tpu as pltpu
from jax.experimental.pallas import tpu_sc as plsc

B, L, S, D, W = 16, 13, 512, 768, 256
SB = 256

NC, NS = 2, 16
DH = D // NC
CH = 32
NCH = S // CH
VL = 16


def _mix_body(w_ref, x_ref, att_ref):
    wv = w_ref[0, :]
    wv = wv - jnp.max(wv)
    ev = jnp.exp(wv)
    wn = ev / jnp.sum(ev)

    acc = x_ref[0, 0] * wn[0]
    for l in range(1, L):
        acc = acc + x_ref[0, l] * wn[l]
    att_ref[0] = acc


def _tc_mix(w2, bpe):
    return pl.pallas_call(
        _mix_body,
        grid=(B, S // SB),
        in_specs=[
            pl.BlockSpec((1, L), lambda b, s: (0, 0)),
            pl.BlockSpec((1, L, SB, D), lambda b, s: (b, 0, s, 0)),
        ],
        out_specs=pl.BlockSpec((1, SB, D), lambda b, s: (b, s, 0)),
        out_shape=jax.ShapeDtypeStruct((B, S, D), jnp.float32),
        compiler_params=pltpu.CompilerParams(
            dimension_semantics=("parallel", "arbitrary")),
    )(w2, bpe)


@functools.partial(
    pl.kernel,
    out_type=jax.ShapeDtypeStruct((B * W, D), jnp.float32),
    mesh=plsc.VectorSubcoreMesh(core_axis_name="c", subcore_axis_name="s"),
    scratch_types=[
        pltpu.VMEM((CH, DH), jnp.float32),
        pltpu.VMEM((CH, DH), jnp.float32),
        pltpu.VMEM((W, DH), jnp.float32),
        pltpu.VMEM((S,), jnp.int32),
        pltpu.SemaphoreType.DMA,
        pltpu.SemaphoreType.DMA,
    ],
)
def _sc_segsum(att_hbm, ids_hbm, out_hbm, row0_v, row1_v, acc_v, ids_v,
               sem0, sem1):
    h = lax.axis_index("c")
    b = lax.axis_index("s")
    hcol = h * DH

    pltpu.sync_copy(ids_hbm.at[pl.ds(b * S, S)], ids_v)

    zv = jnp.zeros((VL,), jnp.float32)

    def zero_row(w, _):
        for j in range(DH // VL):
            acc_v[w, pl.ds(j * VL, VL)] = zv
        return 0

    lax.fori_loop(0, W, zero_row, 0, unroll=False)

    bufs = (row0_v, row1_v)
    sems = (sem0, sem1)

    def fetch(ch, buf, sem):
        return pltpu.async_copy(
            att_hbm.at[pl.ds(b * S + ch * CH, CH), pl.ds(hcol, DH)], buf, sem)

    def drain(ch, buf, sem):
        pltpu.make_async_copy(
            att_hbm.at[pl.ds(b * S + ch * CH, CH), pl.ds(hcol, DH)],
            buf, sem).wait()

    def accum_chunk(ch, buf):
        def group(g, _):
            wvec = ids_v[pl.ds(ch * CH + g * VL, VL)]
            for r16 in range(VL):
                w = wvec[r16]
                r = g * VL + r16
                for j in range(DH // VL):
                    sl = pl.ds(j * VL, VL)
                    acc_v[w, sl] = acc_v[w, sl] + buf[r, sl]
            return 0

        lax.fori_loop(0, CH // VL, group, 0, unroll=False)

    fetch(0, bufs[0], sems[0])
    fetch(1, bufs[1], sems[1])

    def ring(ch2, _):
        for k in range(2):
            ch = ch2 * 2 + k
            drain(ch, bufs[k], sems[k])
            accum_chunk(ch, bufs[k])
            nxt = ch + 2

            @pl.when(nxt < NCH)
            def _():
                fetch(nxt, bufs[k], sems[k])
        return 0

    lax.fori_loop(0, NCH // 2, ring, 0, unroll=False)

    pltpu.sync_copy(acc_v, out_hbm.at[pl.ds(b * W, W), pl.ds(hcol, DH)])


def kernel(bpe_features, word_ids, layer_w):
    w2 = layer_w.reshape(1, L)
    att = _tc_mix(w2, bpe_features).reshape(B * S, D)
    ids = word_ids.reshape(B * S)
    out = _sc_segsum(att, ids)
    return out.reshape(B, W, D)


# speedup vs baseline: 1.3072x; 1.3072x over previous
import functools

import jax
import jax.numpy as jnp
from jax import lax
from jax.experimental import pallas as pl
from jax.experimental.pallas import tpu as pltpu
from jax.experimental.pallas import tpu_sc as plsc

B, L, S, D, W = 16, 13, 512, 768, 256
SB = 256

NC, NS = 2, 16
DH = D // NC
CH = 32
NCH = S // CH
VL = 16
NJ = DH // VL


def _mix_body(w_ref, x_ref, att_ref):
    wv = w_ref[0, :]
    wv = wv - jnp.max(wv)
    ev = jnp.exp(wv)
    wn = ev / jnp.sum(ev)

    acc = x_ref[0, 0] * wn[0]
    for l in range(1, L):
        acc = acc + x_ref[0, l] * wn[l]
    att_ref[0, 0] = acc[:, :DH]
    att_ref[1, 0] = acc[:, DH:]


def _tc_mix(w2, bpe):
    return pl.pallas_call(
        _mix_body,
        grid=(B, S // SB),
        in_specs=[
            pl.BlockSpec((1, L), lambda b, s: (0, 0)),
            pl.BlockSpec((1, L, SB, D), lambda b, s: (b, 0, s, 0)),
        ],
        out_specs=pl.BlockSpec((NC, 1, SB, DH), lambda b, s: (0, b, s, 0)),
        out_shape=jax.ShapeDtypeStruct((NC, B, S, DH), jnp.float32),
        compiler_params=pltpu.CompilerParams(
            dimension_semantics=("parallel", "arbitrary")),
    )(w2, bpe)


@functools.partial(
    pl.kernel,
    out_type=jax.ShapeDtypeStruct((B * W, D), jnp.float32),
    mesh=plsc.VectorSubcoreMesh(core_axis_name="c", subcore_axis_name="s"),
    scratch_types=[
        pltpu.VMEM((CH, DH), jnp.float32),
        pltpu.VMEM((CH, DH), jnp.float32),
        pltpu.VMEM((W, DH), jnp.float32),
        pltpu.VMEM((S,), jnp.int32),
        pltpu.SemaphoreType.DMA,
        pltpu.SemaphoreType.DMA,
    ],
)
def _sc_segsum(att_hbm, ids_hbm, out_hbm, row0_v, row1_v, acc_v, ids_v,
               sem0, sem1):
    h = lax.axis_index("c")
    b = lax.axis_index("s")
    abase = h * (B * S) + b * S

    pltpu.sync_copy(ids_hbm.at[pl.ds(b * S, S)], ids_v)

    zv = jnp.zeros((VL,), jnp.float32)

    def zero_row(w, _):
        for j in range(NJ):
            acc_v[w, pl.ds(j * VL, VL)] = zv
        return 0

    lax.fori_loop(0, W, zero_row, 0, unroll=False)

    bufs = (row0_v, row1_v)
    sems = (sem0, sem1)

    def fetch(ch, buf, sem):
        return pltpu.async_copy(att_hbm.at[pl.ds(abase + ch * CH, CH)], buf,
                                sem)

    def drain(ch, buf, sem):
        pltpu.make_async_copy(att_hbm.at[pl.ds(abase + ch * CH, CH)], buf,
                              sem).wait()

    def accum_chunk(ch, buf, carry):
        def group(g, carry):
            w_prev, acc = carry
            wvec = ids_v[pl.ds(ch * CH + g * VL, VL)]
            for r16 in range(VL):
                w = wvec[r16]
                r = g * VL + r16
                new_seg = w != w_prev
                acc = tuple(
                    jnp.where(new_seg, buf[r, pl.ds(j * VL, VL)],
                              acc[j] + buf[r, pl.ds(j * VL, VL)])
                    for j in range(NJ))
                for j in range(NJ):
                    acc_v[w, pl.ds(j * VL, VL)] = acc[j]
                w_prev = w
            return w_prev, acc

        return lax.fori_loop(0, CH // VL, group, carry, unroll=False)

    fetch(0, bufs[0], sems[0])
    fetch(1, bufs[1], sems[1])

    carry0 = (jnp.int32(-1), tuple(zv for _ in range(NJ)))

    def ring(ch2, carry):
        for k in range(2):
            ch = ch2 * 2 + k
            drain(ch, bufs[k], sems[k])
            carry = accum_chunk(ch, bufs[k], carry)
            nxt = ch + 2

            @pl.when(nxt < NCH)
            def _():
                fetch(nxt, bufs[k], sems[k])
        return carry

    lax.fori_loop(0, NCH // 2, ring, carry0, unroll=False)

    pltpu.sync_copy(acc_v, out_hbm.at[pl.ds(b * W, W), pl.ds(h * DH, DH)])


def kernel(bpe_features, word_ids, layer_w):
    w2 = layer_w.reshape(1, L)
    att = _tc_mix(w2, bpe_features).reshape(NC * B * S, DH)
    ids = word_ids.reshape(B * S)
    out = _sc_segsum(att, ids)
    return out.reshape(B, W, D)
